# baseline (speedup 1.0000x reference)
import jax
import jax.numpy as jnp
from jax import lax
from jax.experimental import pallas as pl
from jax.experimental.pallas import tpu as pltpu
from jax.experimental.pallas import tpu_sc as plsc

N_NODES = 10000
IN_CH = 256
HID_CH = 512
OUT_CH = 256
K_HOPS = 10
ALPHA = 0.1
N_EDGES = 160000

NC = 2
NS = 16
CH = OUT_CH // NC
N_PAD = 10240
RPT = N_PAD // NS
ROW_CHUNKS = RPT // 128
E_TOT = N_EDGES + N_NODES
EC = 84
EPT = EC * 128
E_PAD = NS * EPT
PAD_ROW = N_NODES


def _mlp_body(x_ref, w1_ref, w2_ref, o_ref):
    h = lax.dot_general(x_ref[...], w1_ref[...],
                        dimension_numbers=(((1,), (1,)), ((), ())),
                        preferred_element_type=jnp.float32)
    h = jnp.maximum(h, 0.0)
    o_ref[...] = lax.dot_general(h, w2_ref[...],
                                 dimension_numbers=(((1,), (1,)), ((), ())),
                                 preferred_element_type=jnp.float32)


def _mlp(x, W1, W2):
    grid = 10
    blk = N_NODES // grid
    return pl.pallas_call(
        _mlp_body,
        grid=(grid,),
        in_specs=[
            pl.BlockSpec((blk, IN_CH), lambda i: (i, 0)),
            pl.BlockSpec((HID_CH, IN_CH), lambda i: (0, 0)),
            pl.BlockSpec((OUT_CH, HID_CH), lambda i: (0, 0)),
        ],
        out_specs=pl.BlockSpec((blk, OUT_CH), lambda i: (i, 0)),
        out_shape=jax.ShapeDtypeStruct((N_NODES, OUT_CH), jnp.float32),
    )(x, W1, W2)


def _rsqrt_newton(d):
    i = lax.bitcast_convert_type(d, jnp.int32)
    i = jnp.full((16,), 0x5F3759DF, jnp.int32) - lax.shift_right_arithmetic(
        i, jnp.full((16,), 1, jnp.int32))
    y = lax.bitcast_convert_type(i, jnp.float32)
    for _ in range(4):
        y = y * (1.5 - 0.5 * d * y * y)
    return y


def _prop_body(h_hbm, src_hbm, dst_hbm, out_hbm, y_hbm,
               agg_sh, rows_a, rows_b, src_i0, src_i1, src_i2, src_i3,
               dst_i0, dst_i1, dst_i2, dst_i3, dis_v,
               sem_rows, sem_idx, sem_sc):
    c = lax.axis_index("c")
    s = lax.axis_index("s")
    row0 = s * RPT
    tab0 = c * N_PAD
    erow0 = s * EC
    z16f = jnp.zeros((16,), jnp.float32)
    ones16 = jnp.ones((16,), jnp.float32)
    lane = lax.iota(jnp.int32, 16)
    off16 = jnp.full((16,), tab0, jnp.int32)
    rbufs = (rows_a, rows_b)
    sbufs = (src_i0, src_i1, src_i2, src_i3)
    dbufs = (dst_i0, dst_i1, dst_i2, dst_i3)

    def fill(buf, val):
        @pl.loop(0, 128)
        def _(r):
            for i in range(8):
                buf[r, pl.ds(i * 16, 16)] = val

    fill(rows_b, z16f)

    @pl.loop(0, ROW_CHUNKS)
    def _(k):
        pltpu.sync_copy(rows_b, agg_sh.at[pl.ds(row0 + k * 128, 128)])

    plsc.subcore_barrier()

    fill(rows_a, ones16)

    @pl.loop(0, EC)
    def _(j):
        pltpu.sync_copy(dst_hbm.at[pl.ds(erow0 + j, 1)], dst_i0)
        pltpu.sync_copy(rows_a, agg_sh.at[dst_i0.at[0]], add=True)

    plsc.subcore_barrier()

    @pl.loop(0, ROW_CHUNKS)
    def _(k):
        pltpu.sync_copy(agg_sh.at[pl.ds(row0 + k * 128, 128)], rows_a)

        @pl.loop(0, 8)
        def _(g):
            d = z16f
            for j in range(16):
                v = rows_a[g * 16 + j, pl.ds(0, 16)]
                d = jnp.where(lane == j, v, d)
            y = _rsqrt_newton(d)
            gid = jnp.full((16,), row0 + k * 128 + g * 16, jnp.int32) + lane
            dis_v[pl.ds(k * 128 + g * 16, 16)] = jnp.where(
                gid < N_NODES, y, 0.0)

        pltpu.sync_copy(rows_b, agg_sh.at[pl.ds(row0 + k * 128, 128)])

    @pl.loop(0, ROW_CHUNKS)
    def _(k):
        base = row0 + k * 128
        pltpu.sync_copy(h_hbm.at[pl.ds(tab0 + base, 128)], rows_b)

        @pl.loop(0, 8)
        def _(g):
            dv = dis_v[pl.ds(k * 128 + g * 16, 16)]
            for rr in range(16):
                r = g * 16 + rr
                dr = jnp.full((16,), dv[rr], jnp.float32)
                for i in range(8):
                    sl = pl.ds(i * 16, 16)
                    rows_b[r, sl] = rows_b[r, sl] * dr

        pltpu.sync_copy(rows_b, y_hbm.at[pl.ds(tab0 + base, 128)])

    plsc.subcore_barrier()

    def fetch_idx(j, p):
        pltpu.async_copy(src_hbm.at[pl.ds(erow0 + j, 1)], sbufs[p], sem_idx)
        pltpu.async_copy(dst_hbm.at[pl.ds(erow0 + j, 1)], dbufs[p], sem_idx)

    def wait_idx(p):
        pltpu.make_async_copy(
            src_hbm.at[pl.ds(0, 1)], sbufs[p], sem_idx).wait()
        pltpu.make_async_copy(
            dst_hbm.at[pl.ds(0, 1)], dbufs[p], sem_idx).wait()

    def offset_src(p):
        for i in range(8):
            sl = pl.ds(i * 16, 16)
            sbufs[p][0, sl] = sbufs[p][0, sl] + off16

    def start_gather2(b, p):
        pltpu.async_copy(y_hbm.at[sbufs[p].at[0]], rbufs[b], sem_rows)

    def start_gather(p):
        start_gather2(p, p)

    def wait_gather(b):
        pltpu.make_async_copy(
            y_hbm.at[pl.ds(0, 128)], rbufs[b], sem_rows).wait()

    def wait_scatter():
        pltpu.make_async_copy(
            rbufs[0], agg_sh.at[pl.ds(0, 128)], sem_sc).wait()

    def scatter_phase():
        fetch_idx(0, 0)
        fetch_idx(1, 1)
        fetch_idx(2, 2)
        wait_idx(0)
        offset_src(0)
        start_gather(0)

        @pl.loop(0, EC, step=4)
        def _(j):
            for q in range(4):
                jj = j + q
                b = q % 2
                wait_gather(b)
                pltpu.async_copy(rbufs[b], agg_sh.at[dbufs[q].at[0]],
                                 sem_sc, add=True)

                @pl.when(jj + 1 < EC)
                def _():
                    wait_idx((q + 1) % 4)
                    offset_src((q + 1) % 4)

                @pl.when(jj >= 1)
                def _():
                    wait_scatter()

                @pl.when(jj + 1 < EC)
                def _():
                    start_gather2(1 - b, (q + 1) % 4)

                @pl.when(jj + 3 < EC)
                def _():
                    fetch_idx(jj + 3, (q + 3) % 4)

        wait_scatter()

    def update_phase(dst_tab, last):
        @pl.loop(0, ROW_CHUNKS)
        def _(k):
            base = row0 + k * 128
            pltpu.sync_copy(agg_sh.at[pl.ds(base, 128)], rows_a)
            pltpu.sync_copy(h_hbm.at[pl.ds(tab0 + base, 128)], rows_b)

            @pl.loop(0, 8)
            def _(g):
                dv = dis_v[pl.ds(k * 128 + g * 16, 16)]
                for rr in range(16):
                    r = g * 16 + rr
                    dr = dv[rr]
                    if last:
                        av = jnp.full((16,), (1.0 - ALPHA) * dr, jnp.float32)
                        bv = jnp.full((16,), ALPHA, jnp.float32)
                    else:
                        av = jnp.full((16,), (1.0 - ALPHA) * dr * dr,
                                      jnp.float32)
                        bv = jnp.full((16,), ALPHA * dr, jnp.float32)
                    for i in range(8):
                        sl = pl.ds(i * 16, 16)
                        rows_b[r, sl] = (rows_a[r, sl] * av
                                         + rows_b[r, sl] * bv)
                        rows_a[r, sl] = z16f

            pltpu.sync_copy(rows_a, agg_sh.at[pl.ds(base, 128)])
            pltpu.sync_copy(rows_b, dst_tab.at[pl.ds(tab0 + base, 128)])

        plsc.subcore_barrier()

    @pl.loop(0, K_HOPS - 1)
    def _(hop):
        scatter_phase()
        plsc.subcore_barrier()
        update_phase(y_hbm, last=False)

    scatter_phase()
    plsc.subcore_barrier()
    update_phase(out_hbm, last=True)


def _propagate(h_flat, src_t, dst_t):
    mesh = plsc.VectorSubcoreMesh(core_axis_name="c", subcore_axis_name="s")
    run = pl.kernel(
        _prop_body,
        out_type=(
            jax.ShapeDtypeStruct((NC * N_PAD, CH), jnp.float32),
            jax.ShapeDtypeStruct((NC * N_PAD, CH), jnp.float32),
        ),
        mesh=mesh,
        scratch_types=[
            pltpu.VMEM_SHARED((N_PAD, CH), jnp.float32),
            pltpu.VMEM((128, CH), jnp.float32),
            pltpu.VMEM((128, CH), jnp.float32),
            pltpu.VMEM((1, 128), jnp.int32),
            pltpu.VMEM((1, 128), jnp.int32),
            pltpu.VMEM((1, 128), jnp.int32),
            pltpu.VMEM((1, 128), jnp.int32),
            pltpu.VMEM((1, 128), jnp.int32),
            pltpu.VMEM((1, 128), jnp.int32),
            pltpu.VMEM((1, 128), jnp.int32),
            pltpu.VMEM((1, 128), jnp.int32),
            pltpu.VMEM((RPT,), jnp.float32),
            pltpu.SemaphoreType.DMA,
            pltpu.SemaphoreType.DMA,
            pltpu.SemaphoreType.DMA,
        ],
    )
    out_flat, _ = run(h_flat, src_t, dst_t)
    return out_flat


def kernel(x, edge_idx, W1, W2):
    h = _mlp(x, W1, W2)

    pad = ((0, N_PAD - N_NODES), (0, 0))
    h_flat = jnp.concatenate(
        [jnp.pad(h[:, :CH], pad), jnp.pad(h[:, CH:], pad)], axis=0)

    src = edge_idx[0].astype(jnp.int32)
    dst = edge_idx[1].astype(jnp.int32)
    loop = jnp.arange(N_NODES, dtype=jnp.int32)
    fill = jnp.full((E_PAD - E_TOT,), PAD_ROW, jnp.int32)
    src_t = jnp.concatenate([src, loop, fill]).reshape(NS * EC, 128)
    dst_t = jnp.concatenate([dst, loop, fill]).reshape(NS * EC, 128)

    out_flat = _propagate(h_flat, src_t, dst_t)
    return jnp.concatenate(
        [out_flat[:N_NODES], out_flat[N_PAD:N_PAD + N_NODES]], axis=1)


# speedup vs baseline: 1.0179x; 1.0179x over previous
import jax
import jax.numpy as jnp
from jax import lax
from jax.experimental import pallas as pl
from jax.experimental.pallas import tpu as pltpu
from jax.experimental.pallas import tpu_sc as plsc

N_NODES = 10000
IN_CH = 256
HID_CH = 512
OUT_CH = 256
K_HOPS = 10
ALPHA = 0.1
N_EDGES = 160000

NC = 2
NS = 16
CH = OUT_CH // NC
N_PAD = 10240
RPT = N_PAD // NS
N_AGG = 10112
UCH = 64
E_TOT = N_EDGES + N_NODES
CHUNK = 112
EC = 96
EPT = EC * CHUNK
E_PAD = NS * EPT
PAD_ROW = N_NODES


def _mlp_body(x_ref, w1_ref, w2_ref, o_ref):
    h = lax.dot_general(x_ref[...], w1_ref[...],
                        dimension_numbers=(((1,), (1,)), ((), ())),
                        preferred_element_type=jnp.float32)
    h = jnp.maximum(h, 0.0)
    o_ref[...] = lax.dot_general(h, w2_ref[...],
                                 dimension_numbers=(((1,), (1,)), ((), ())),
                                 preferred_element_type=jnp.float32)


def _mlp(x, W1, W2):
    grid = 10
    blk = N_NODES // grid
    return pl.pallas_call(
        _mlp_body,
        grid=(grid,),
        in_specs=[
            pl.BlockSpec((blk, IN_CH), lambda i: (i, 0)),
            pl.BlockSpec((HID_CH, IN_CH), lambda i: (0, 0)),
            pl.BlockSpec((OUT_CH, HID_CH), lambda i: (0, 0)),
        ],
        out_specs=pl.BlockSpec((blk, OUT_CH), lambda i: (i, 0)),
        out_shape=jax.ShapeDtypeStruct((N_NODES, OUT_CH), jnp.float32),
    )(x, W1, W2)


def _rsqrt_newton(d):
    i = lax.bitcast_convert_type(d, jnp.int32)
    i = jnp.full((16,), 0x5F3759DF, jnp.int32) - lax.shift_right_arithmetic(
        i, jnp.full((16,), 1, jnp.int32))
    y = lax.bitcast_convert_type(i, jnp.float32)
    for _ in range(4):
        y = y * (1.5 - 0.5 * d * y * y)
    return y


def _prop_body(h_hbm, e_hbm, out_hbm, y_hbm,
               agg_sh, rows_0, rows_1, rows_2,
               i0, i1, i2, i3, i4, i5, dis_v,
               sem_rows, sem_idx, sem_sc):
    c = lax.axis_index("c")
    s = lax.axis_index("s")
    row0 = s * RPT
    tab0 = c * N_PAD
    erow0 = s * EC
    nct = jnp.where(s == NS - 1, (N_AGG - (NS - 1) * RPT) // UCH, RPT // UCH)
    z16f = jnp.zeros((16,), jnp.float32)
    ones16 = jnp.ones((16,), jnp.float32)
    lane = lax.iota(jnp.int32, 16)
    off16 = jnp.full((16,), tab0, jnp.int32)
    rbufs = (rows_0, rows_1, rows_2)
    ibufs = (i0, i1, i2, i3, i4, i5)

    def fill(buf, n, val):
        @pl.loop(0, n)
        def _(r):
            for i in range(8):
                buf[r, pl.ds(i * 16, 16)] = val

    fill(rows_2, UCH, z16f)

    @pl.loop(0, nct)
    def _(k):
        pltpu.sync_copy(rows_2.at[pl.ds(0, UCH)],
                        agg_sh.at[pl.ds(row0 + k * UCH, UCH)])

    plsc.subcore_barrier()

    fill(rows_0, CHUNK, ones16)

    @pl.loop(0, EC)
    def _(j):
        pltpu.sync_copy(e_hbm.at[pl.ds(2 * (erow0 + j) + 1, 1)],
                        i0.at[pl.ds(0, 1)])
        pltpu.sync_copy(rows_0, agg_sh.at[i0.at[0]], add=True)

    plsc.subcore_barrier()

    @pl.loop(0, nct)
    def _(k):
        base = row0 + k * UCH
        pltpu.sync_copy(agg_sh.at[pl.ds(base, UCH)],
                        rows_1.at[pl.ds(0, UCH)])

        @pl.loop(0, UCH // 16)
        def _(g):
            d = z16f
            for j in range(16):
                v = rows_1[g * 16 + j, pl.ds(0, 16)]
                d = jnp.where(lane == j, v, d)
            y = _rsqrt_newton(d)
            gid = jnp.full((16,), base + g * 16, jnp.int32) + lane
            dis_v[pl.ds(k * UCH + g * 16, 16)] = jnp.where(
                gid < N_NODES, y, 0.0)

        pltpu.sync_copy(rows_2.at[pl.ds(0, UCH)],
                        agg_sh.at[pl.ds(base, UCH)])

    @pl.loop(0, nct)
    def _(k):
        base = row0 + k * UCH
        pltpu.sync_copy(h_hbm.at[pl.ds(tab0 + base, UCH)],
                        rows_1.at[pl.ds(0, UCH)])

        @pl.loop(0, UCH // 16)
        def _(g):
            dv = dis_v[pl.ds(k * UCH + g * 16, 16)]
            for rr in range(16):
                r = g * 16 + rr
                dr = jnp.full((16,), dv[rr], jnp.float32)
                for i in range(8):
                    sl = pl.ds(i * 16, 16)
                    rows_1[r, sl] = rows_1[r, sl] * dr

        pltpu.sync_copy(rows_1.at[pl.ds(0, UCH)],
                        y_hbm.at[pl.ds(tab0 + base, UCH)])

    plsc.subcore_barrier()

    def fetch_idx(j, p):
        pltpu.async_copy(e_hbm.at[pl.ds(2 * (erow0 + j), 2)],
                         ibufs[p], sem_idx)

    def wait_idx(p):
        pltpu.make_async_copy(e_hbm.at[pl.ds(0, 2)], ibufs[p],
                              sem_idx).wait()

    def offset_src(p):
        for i in range(7):
            sl = pl.ds(i * 16, 16)
            ibufs[p][0, sl] = ibufs[p][0, sl] + off16

    def start_gather(b, p):
        pltpu.async_copy(y_hbm.at[ibufs[p].at[0]], rbufs[b], sem_rows)

    def wait_gather(b):
        pltpu.make_async_copy(
            y_hbm.at[pl.ds(0, CHUNK)], rbufs[b], sem_rows).wait()

    def wait_scatter():
        pltpu.make_async_copy(
            rbufs[0], agg_sh.at[pl.ds(0, CHUNK)], sem_sc).wait()

    def scatter_phase():
        fetch_idx(0, 0)
        fetch_idx(1, 1)
        fetch_idx(2, 2)
        wait_idx(0)
        offset_src(0)
        start_gather(0, 0)
        wait_idx(1)
        offset_src(1)
        start_gather(1, 1)

        @pl.loop(0, EC, step=6)
        def _(j):
            for q in range(6):
                jj = j + q
                b = q % 3
                wait_gather(b)
                pltpu.async_copy(rbufs[b], agg_sh.at[ibufs[q].at[1]],
                                 sem_sc, add=True)

                @pl.when(jj >= 2)
                def _():
                    wait_scatter()

                @pl.when(jj + 2 < EC)
                def _():
                    wait_idx((q + 2) % 6)
                    offset_src((q + 2) % 6)
                    start_gather((q + 2) % 3, (q + 2) % 6)

                @pl.when(jj + 3 < EC)
                def _():
                    fetch_idx(jj + 3, (q + 3) % 6)

        wait_scatter()
        wait_scatter()

    def update_phase(dst_tab, last):
        @pl.loop(0, nct)
        def _(k):
            base = row0 + k * UCH
            pltpu.sync_copy(agg_sh.at[pl.ds(base, UCH)],
                            rows_0.at[pl.ds(0, UCH)])
            pltpu.sync_copy(h_hbm.at[pl.ds(tab0 + base, UCH)],
                            rows_1.at[pl.ds(0, UCH)])

            @pl.loop(0, UCH // 16)
            def _(g):
                dv = dis_v[pl.ds(k * UCH + g * 16, 16)]
                for rr in range(16):
                    r = g * 16 + rr
                    dr = dv[rr]
                    if last:
                        av = jnp.full((16,), (1.0 - ALPHA) * dr, jnp.float32)
                        bv = jnp.full((16,), ALPHA, jnp.float32)
                    else:
                        av = jnp.full((16,), (1.0 - ALPHA) * dr * dr,
                                      jnp.float32)
                        bv = jnp.full((16,), ALPHA * dr, jnp.float32)
                    for i in range(8):
                        sl = pl.ds(i * 16, 16)
                        rows_1[r, sl] = (rows_0[r, sl] * av
                                         + rows_1[r, sl] * bv)
                        rows_0[r, sl] = z16f

            pltpu.sync_copy(rows_0.at[pl.ds(0, UCH)],
                            agg_sh.at[pl.ds(base, UCH)])
            pltpu.sync_copy(rows_1.at[pl.ds(0, UCH)],
                            dst_tab.at[pl.ds(tab0 + base, UCH)])

        plsc.subcore_barrier()

    @pl.loop(0, K_HOPS - 1)
    def _(hop):
        scatter_phase()
        plsc.subcore_barrier()
        update_phase(y_hbm, last=False)

    scatter_phase()
    plsc.subcore_barrier()
    update_phase(out_hbm, last=True)


def _propagate(h_flat, e2):
    mesh = plsc.VectorSubcoreMesh(core_axis_name="c", subcore_axis_name="s")
    run = pl.kernel(
        _prop_body,
        out_type=(
            jax.ShapeDtypeStruct((NC * N_PAD, CH), jnp.float32),
            jax.ShapeDtypeStruct((NC * N_PAD, CH), jnp.float32),
        ),
        mesh=mesh,
        scratch_types=[
            pltpu.VMEM_SHARED((N_AGG, CH), jnp.float32),
            pltpu.VMEM((CHUNK, CH), jnp.float32),
            pltpu.VMEM((CHUNK, CH), jnp.float32),
            pltpu.VMEM((CHUNK, CH), jnp.float32),
            pltpu.VMEM((2, CHUNK), jnp.int32),
            pltpu.VMEM((2, CHUNK), jnp.int32),
            pltpu.VMEM((2, CHUNK), jnp.int32),
            pltpu.VMEM((2, CHUNK), jnp.int32),
            pltpu.VMEM((2, CHUNK), jnp.int32),
            pltpu.VMEM((2, CHUNK), jnp.int32),
            pltpu.VMEM((RPT,), jnp.float32),
            pltpu.SemaphoreType.DMA,
            pltpu.SemaphoreType.DMA,
            pltpu.SemaphoreType.DMA,
        ],
    )
    out_flat, _ = run(h_flat, e2)
    return out_flat


def kernel(x, edge_idx, W1, W2):
    h = _mlp(x, W1, W2)

    pad = ((0, N_PAD - N_NODES), (0, 0))
    h_flat = jnp.concatenate(
        [jnp.pad(h[:, :CH], pad), jnp.pad(h[:, CH:], pad)], axis=0)

    src = edge_idx[0].astype(jnp.int32)
    dst = edge_idx[1].astype(jnp.int32)
    loop = jnp.arange(N_NODES, dtype=jnp.int32)
    fill = jnp.full((E_PAD - E_TOT,), PAD_ROW, jnp.int32)
    sp = jnp.concatenate([src, loop, fill]).reshape(NS * EC, CHUNK)
    dp = jnp.concatenate([dst, loop, fill]).reshape(NS * EC, CHUNK)
    e2 = jnp.stack([sp, dp], axis=1).reshape(2 * NS * EC, CHUNK)

    out_flat = _propagate(h_flat, e2)
    return jnp.concatenate(
        [out_flat[:N_NODES], out_flat[N_PAD:N_PAD + N_NODES]], axis=1)
